# baseline (speedup 1.0000x reference)
import math

import jax
import jax.numpy as jnp
from jax import lax
from jax.experimental import pallas as pl
from jax.experimental.pallas import tpu as pltpu
from jax.experimental.pallas import tpu_sc as plsc

N = 10000
E = 160000
D = 128
DA = 16
EPS = 1e-09
NORM = 1.0 / math.sqrt(20.0)

TN = 400
TE = 256
C = 128
NW = 32
NS = 16
SPLITS = [19968] * 7 + [20224]
ROWS_PER_SUB = 624
ROWS_TAIL = N - NS * ROWS_PER_SUB


def _rownorm(y):
    m = jnp.mean(y, axis=1, keepdims=True)
    c = y - m
    var = jnp.sum(c * c, axis=1, keepdims=True) * (1.0 / (D - 1))
    return y / (jnp.sqrt(var) + EPS)


def _silu(z):
    return z * jax.nn.sigmoid(z)


def _stage_a_body(xn_ref, attr_ref, wbt_ref, wlt_ref, bl_ref, out_ref):
    x1 = xn_ref[...]
    x2 = attr_ref[...]
    outer = jnp.concatenate([x2[:, j:j + 1] * x1 for j in range(DA)], axis=1)
    xbi = jnp.dot(outer, wbt_ref[...], preferred_element_type=jnp.float32)
    wlt = wlt_ref[...]
    y = (jnp.dot(x1, wlt[:D], preferred_element_type=jnp.float32)
         + jnp.dot(x2, wlt[D:D + DA], preferred_element_type=jnp.float32)
         + jnp.dot(xbi, wlt[D + DA:], preferred_element_type=jnp.float32)
         + bl_ref[...])
    out_ref[...] = _rownorm(y)


def _make_gather(nch):
    per_w, rag = nch // NW, nch % NW
    npairs, odd = per_w // 2, per_w % 2

    def body(table, srci, dsti, srows, drows,
             sidx0, didx0, sbuf0, dbuf0, sidx1, didx1, sbuf1, dbuf1,
             semA, semB, semC, semD, semE, semF):
        c = lax.axis_index("c")
        s = lax.axis_index("s")
        wid = s * 2 + c

        def one(base, sidx, didx, sbuf, dbuf, semi, semg, semo):
            ci = pltpu.async_copy(srci.at[pl.ds(base, C)], sidx, semi)
            cj = pltpu.async_copy(dsti.at[pl.ds(base, C)], didx, semi)
            ci.wait()
            cj.wait()
            g1 = pltpu.async_copy(table.at[sidx], sbuf, semg)
            g2 = pltpu.async_copy(table.at[didx], dbuf, semg)
            g1.wait()
            g2.wait()
            o1 = pltpu.async_copy(sbuf, srows.at[pl.ds(base, C)], semo)
            o2 = pltpu.async_copy(dbuf, drows.at[pl.ds(base, C)], semo)
            o1.wait()
            o2.wait()

        def pair(i, carry):
            ba = (wid + NW * (2 * i)) * C
            bb = (wid + NW * (2 * i + 1)) * C
            ia1 = pltpu.async_copy(srci.at[pl.ds(ba, C)], sidx0, semA)
            ia2 = pltpu.async_copy(dsti.at[pl.ds(ba, C)], didx0, semA)
            ib1 = pltpu.async_copy(srci.at[pl.ds(bb, C)], sidx1, semB)
            ib2 = pltpu.async_copy(dsti.at[pl.ds(bb, C)], didx1, semB)
            ia1.wait()
            ia2.wait()
            ga1 = pltpu.async_copy(table.at[sidx0], sbuf0, semC)
            ga2 = pltpu.async_copy(table.at[didx0], dbuf0, semC)
            ib1.wait()
            ib2.wait()
            gb1 = pltpu.async_copy(table.at[sidx1], sbuf1, semD)
            gb2 = pltpu.async_copy(table.at[didx1], dbuf1, semD)
            ga1.wait()
            ga2.wait()
            oa1 = pltpu.async_copy(sbuf0, srows.at[pl.ds(ba, C)], semE)
            oa2 = pltpu.async_copy(dbuf0, drows.at[pl.ds(ba, C)], semE)
            gb1.wait()
            gb2.wait()
            ob1 = pltpu.async_copy(sbuf1, srows.at[pl.ds(bb, C)], semF)
            ob2 = pltpu.async_copy(dbuf1, drows.at[pl.ds(bb, C)], semF)
            oa1.wait()
            oa2.wait()
            ob1.wait()
            ob2.wait()
            return carry

        lax.fori_loop(0, npairs, pair, 0)
        if odd:
            one((wid + NW * (2 * npairs)) * C, sidx0, didx0, sbuf0, dbuf0,
                semA, semC, semE)
        if rag:
            @pl.when(wid < rag)
            def _tail():
                one((wid + NW * per_w) * C, sidx1, didx1, sbuf1, dbuf1,
                    semB, semD, semF)

    return body


def _stage_c_body(src_ref, dst_ref, a_ref, fc1_ref, bfc1_ref, fc2_ref,
                  bfc2_ref, wb2_ref, wl1_ref, bl1_ref, wbxe_ref, wl2_ref,
                  bl2_ref, out_ref):
    sT = src_ref[...].T
    dT = dst_ref[...].T
    aT = a_ref[...]
    wT = _silu(fc1_ref[...] * aT + bfc1_ref[...])
    gradT = wT * (sT - dT)
    aveT = wT * (sT + dT) * 0.5
    xbiT = jnp.zeros((D, TE), jnp.float32)
    for j0 in range(0, D, 32):
        outerT = jnp.concatenate(
            [(aveT[j:j + 1, :] * gradT).astype(jnp.bfloat16)
             for j in range(j0, j0 + 32)], axis=0)
        xbiT = xbiT + jnp.dot(wb2_ref[:, pl.ds(j0 * D, 32 * D)], outerT,
                              preferred_element_type=jnp.float32)
    wl1 = wl1_ref[...]
    xeT = (jnp.dot(wl1[:, :D], gradT, preferred_element_type=jnp.float32)
           + jnp.dot(wl1[:, D:2 * D], aveT, preferred_element_type=jnp.float32)
           + jnp.dot(wl1[:, 2 * D:], xbiT, preferred_element_type=jnp.float32)
           + bl1_ref[...])
    xbi2T = jnp.dot(wbxe_ref[...], xeT, preferred_element_type=jnp.float32) * aT
    wl2 = wl2_ref[...]
    xe2T = (jnp.dot(wl2[:, :D], xeT, preferred_element_type=jnp.float32)
            + wl2[:, D:D + 1] * aT
            + jnp.dot(wl2[:, D + 1:], xbi2T, preferred_element_type=jnp.float32)
            + bl2_ref[...])
    m = jnp.mean(xe2T, axis=0, keepdims=True)
    cen = xe2T - m
    var = jnp.sum(cen * cen, axis=0, keepdims=True) * (1.0 / (D - 1))
    xe2T = xe2T / (jnp.sqrt(var) + EPS)
    w2T = _silu(fc2_ref[...] * aT + bfc2_ref[...])
    out_ref[...] = (w2T * xe2T * NORM).T


def _make_scatter(nch):
    per_s, rag = nch // NS, nch % NS
    npairs, odd = per_s // 2, per_s % 2

    def body(vals, dsti, srci, zrows, xn1, xn2, idxb0, vbuf0, idxb1, vbuf1,
             acc, semA, semB, semC, semD):
        c = lax.axis_index("c")
        s = lax.axis_index("s")
        pltpu.sync_copy(zrows.at[pl.ds(s * ROWS_PER_SUB, ROWS_PER_SUB)],
                        acc.at[pl.ds(s * ROWS_PER_SUB, ROWS_PER_SUB)])

        @pl.when(s == 0)
        def _ztail():
            pltpu.sync_copy(zrows.at[pl.ds(NS * ROWS_PER_SUB, ROWS_TAIL)],
                            acc.at[pl.ds(NS * ROWS_PER_SUB, ROWS_TAIL)])

        plsc.subcore_barrier()

        def run(idx_hbm):
            def one(base, idxb, vbuf, semi, semv):
                ia = pltpu.async_copy(idx_hbm.at[pl.ds(base, C)], idxb, semi)
                va = pltpu.async_copy(vals.at[pl.ds(base, C)], vbuf, semv)
                ia.wait()
                va.wait()
                pltpu.sync_copy(vbuf, acc.at[idxb], add=True)

            def pair(i, carry):
                ba = (s + NS * (2 * i)) * C
                bb = (s + NS * (2 * i + 1)) * C
                ia = pltpu.async_copy(idx_hbm.at[pl.ds(ba, C)], idxb0, semA)
                va = pltpu.async_copy(vals.at[pl.ds(ba, C)], vbuf0, semB)
                ib = pltpu.async_copy(idx_hbm.at[pl.ds(bb, C)], idxb1, semC)
                vb = pltpu.async_copy(vals.at[pl.ds(bb, C)], vbuf1, semD)
                ia.wait()
                va.wait()
                pltpu.sync_copy(vbuf0, acc.at[idxb0], add=True)
                ib.wait()
                vb.wait()
                pltpu.sync_copy(vbuf1, acc.at[idxb1], add=True)
                return carry

            lax.fori_loop(0, npairs, pair, 0)
            if odd:
                one((s + NS * (2 * npairs)) * C, idxb0, vbuf0, semA, semB)
            if rag:
                @pl.when(s < rag)
                def _tail():
                    one((s + NS * per_s) * C, idxb1, vbuf1, semC, semD)

        @pl.when(c == 0)
        def _dst():
            run(dsti)

        @pl.when(c == 1)
        def _src():
            run(srci)

        plsc.subcore_barrier()

        @pl.when(c == 0)
        def _out1():
            pltpu.sync_copy(acc.at[pl.ds(s * ROWS_PER_SUB, ROWS_PER_SUB)],
                            xn1.at[pl.ds(s * ROWS_PER_SUB, ROWS_PER_SUB)])

            @pl.when(s == 0)
            def _t1():
                pltpu.sync_copy(acc.at[pl.ds(NS * ROWS_PER_SUB, ROWS_TAIL)],
                                xn1.at[pl.ds(NS * ROWS_PER_SUB, ROWS_TAIL)])

        @pl.when(c == 1)
        def _out2():
            pltpu.sync_copy(acc.at[pl.ds(s * ROWS_PER_SUB, ROWS_PER_SUB)],
                            xn2.at[pl.ds(s * ROWS_PER_SUB, ROWS_PER_SUB)])

            @pl.when(s == 0)
            def _t2():
                pltpu.sync_copy(acc.at[pl.ds(NS * ROWS_PER_SUB, ROWS_TAIL)],
                                xn2.at[pl.ds(NS * ROWS_PER_SUB, ROWS_TAIL)])

    return body


def _stage_e_body(*refs):
    nparts = len(SPLITS)
    part_refs = refs[:2 * nparts]
    wb2_ref, wl_ref, bl_ref, out_ref = refs[2 * nparts:]
    xn1 = sum(part_refs[2 * k][...] for k in range(nparts))
    xn2 = sum(part_refs[2 * k + 1][...] for k in range(nparts))
    ddT = (xn1 - xn2).T
    smT = (xn1 + xn2).T
    xbiT = jnp.zeros((D, TN), jnp.float32)
    for j0 in range(0, D, 32):
        outerT = jnp.concatenate(
            [(smT[j:j + 1, :] * ddT).astype(jnp.bfloat16)
             for j in range(j0, j0 + 32)], axis=0)
        xbiT = xbiT + jnp.dot(wb2_ref[:, pl.ds(j0 * D, 32 * D)], outerT,
                              preferred_element_type=jnp.float32)
    wl = wl_ref[...]
    yT = (jnp.dot(wl[:, :D], ddT, preferred_element_type=jnp.float32)
          + jnp.dot(wl[:, D:2 * D], smT, preferred_element_type=jnp.float32)
          + jnp.dot(wl[:, 2 * D:], xbiT, preferred_element_type=jnp.float32)
          + bl_ref[...])
    yT = _silu(yT)
    m = jnp.mean(yT, axis=0, keepdims=True)
    cen = yT - m
    var = jnp.sum(cen * cen, axis=0, keepdims=True) * (1.0 / (D - 1))
    out_ref[...] = (yT / (jnp.sqrt(var) + EPS)).T


_GATHER_SCRATCH = [
    pltpu.VMEM((C,), jnp.int32),
    pltpu.VMEM((C,), jnp.int32),
    pltpu.VMEM((C, D), jnp.float32),
    pltpu.VMEM((C, D), jnp.float32),
    pltpu.VMEM((C,), jnp.int32),
    pltpu.VMEM((C,), jnp.int32),
    pltpu.VMEM((C, D), jnp.float32),
    pltpu.VMEM((C, D), jnp.float32),
] + [pltpu.SemaphoreType.DMA] * 6

_SCATTER_SCRATCH = [
    pltpu.VMEM((C,), jnp.int32),
    pltpu.VMEM((C, D), jnp.float32),
    pltpu.VMEM((C,), jnp.int32),
    pltpu.VMEM((C, D), jnp.float32),
    pltpu.VMEM_SHARED((N, D), jnp.float32),
] + [pltpu.SemaphoreType.DMA] * 4


def kernel(xn, xn_attr, xe_attr, xe_src, xe_dst, Wb_xn, Wl_xn, bl_xn,
           W_fc1, b_fc1, Wb_n2e, Wl_n2e, bl_n2e, Wb_xe, Wl_xe, bl_xe,
           W_fc2, b_fc2, Wb_e2n, Wl_e2n, bl_e2n):
    f32 = jnp.float32
    xe_src = xe_src.astype(jnp.int32)
    xe_dst = xe_dst.astype(jnp.int32)

    wbt_xn = jnp.transpose(Wb_xn, (2, 1, 0)).reshape(DA * D, D)
    wb2_n2e = jnp.transpose(Wb_n2e, (0, 2, 1)).reshape(D, D * D).astype(jnp.bfloat16)
    wb2_e2n = jnp.transpose(Wb_e2n, (0, 2, 1)).reshape(D, D * D).astype(jnp.bfloat16)
    wlt_xn = Wl_xn.T
    wbxe0 = Wb_xe[:, :, 0]
    aT_edge = xe_attr.T
    fc1c = W_fc1
    fc2c = W_fc2
    bfc1c = b_fc1.reshape(D, 1)
    bfc2c = b_fc2.reshape(D, 1)
    bl_xn2 = bl_xn.reshape(1, D)
    bl_n2ec = bl_n2e.reshape(D, 1)
    bl_xec = bl_xe.reshape(D, 1)

    xnm = pl.pallas_call(
        _stage_a_body,
        grid=(N // TN,),
        in_specs=[
            pl.BlockSpec((TN, D), lambda i: (i, 0)),
            pl.BlockSpec((TN, DA), lambda i: (i, 0)),
            pl.BlockSpec((DA * D, D), lambda i: (0, 0)),
            pl.BlockSpec((2 * D + DA, D), lambda i: (0, 0)),
            pl.BlockSpec((1, D), lambda i: (0, 0)),
        ],
        out_specs=pl.BlockSpec((TN, D), lambda i: (i, 0)),
        out_shape=jax.ShapeDtypeStruct((N, D), f32),
    )(xn, xn_attr, wbt_xn, wlt_xn, bl_xn2)

    mesh = plsc.VectorSubcoreMesh(core_axis_name="c", subcore_axis_name="s")

    def gather(nch, srci, dsti):
        return pl.kernel(
            _make_gather(nch),
            out_type=[jax.ShapeDtypeStruct((nch * C, D), f32),
                      jax.ShapeDtypeStruct((nch * C, D), f32)],
            mesh=mesh,
            scratch_types=_GATHER_SCRATCH,
        )(xnm, srci, dsti)

    def edge_compute(srows, drows, aT, ne):
        return pl.pallas_call(
            _stage_c_body,
            grid=(ne // TE,),
            in_specs=[
                pl.BlockSpec((TE, D), lambda i: (i, 0)),
                pl.BlockSpec((TE, D), lambda i: (i, 0)),
                pl.BlockSpec((1, TE), lambda i: (0, i)),
                pl.BlockSpec((D, 1), lambda i: (0, 0)),
                pl.BlockSpec((D, 1), lambda i: (0, 0)),
                pl.BlockSpec((D, 1), lambda i: (0, 0)),
                pl.BlockSpec((D, 1), lambda i: (0, 0)),
                pl.BlockSpec((D, D * D), lambda i: (0, 0)),
                pl.BlockSpec((D, 3 * D), lambda i: (0, 0)),
                pl.BlockSpec((D, 1), lambda i: (0, 0)),
                pl.BlockSpec((D, D), lambda i: (0, 0)),
                pl.BlockSpec((D, 2 * D + 1), lambda i: (0, 0)),
                pl.BlockSpec((D, 1), lambda i: (0, 0)),
            ],
            out_specs=pl.BlockSpec((TE, D), lambda i: (i, 0)),
            out_shape=jax.ShapeDtypeStruct((ne, D), f32),
        )(srows, drows, aT, fc1c, bfc1c, fc2c, bfc2c, wb2_n2e, Wl_n2e,
          bl_n2ec, wbxe0, Wl_xe, bl_xec)

    zrows = jnp.zeros((N, D), f32)

    def scatter(nch, vals, dsti, srci):
        return pl.kernel(
            _make_scatter(nch),
            out_type=[jax.ShapeDtypeStruct((N, D), f32),
                      jax.ShapeDtypeStruct((N, D), f32)],
            mesh=mesh,
            scratch_types=_SCATTER_SCRATCH,
        )(vals, dsti, srci, zrows)

    bounds = [0]
    for ne in SPLITS:
        bounds.append(bounds[-1] + ne)
    parts = []
    for k in range(len(SPLITS)):
        lo, ne = bounds[k], SPLITS[k]
        parts.append((xe_src[lo:lo + ne], xe_dst[lo:lo + ne],
                      aT_edge[:, lo:lo + ne], ne))

    rows = [gather(ne // C, s, d) for (s, d, _, ne) in parts]
    vals = [edge_compute(r[0], r[1], a, ne)
            for r, (_, _, a, ne) in zip(rows, parts)]
    sums = [scatter(ne // C, v, d, s)
            for v, (s, d, _, ne) in zip(vals, parts)]

    out = pl.pallas_call(
        _stage_e_body,
        grid=(N // TN,),
        in_specs=(
            [pl.BlockSpec((TN, D), lambda i: (i, 0))] * (2 * len(SPLITS)) + [
                pl.BlockSpec((D, D * D), lambda i: (0, 0)),
                pl.BlockSpec((D, 3 * D), lambda i: (0, 0)),
                pl.BlockSpec((D, 1), lambda i: (0, 0)),
            ]),
        out_specs=pl.BlockSpec((TN, D), lambda i: (i, 0)),
        out_shape=jax.ShapeDtypeStruct((N, D), f32),
    )(*[x for pair in sums for x in pair],
      wb2_e2n, Wl_e2n, bl_e2n.reshape(D, 1))
    return out


# speedup vs baseline: 1.0066x; 1.0066x over previous
import math

import jax
import jax.numpy as jnp
from jax import lax
from jax.experimental import pallas as pl
from jax.experimental.pallas import tpu as pltpu
from jax.experimental.pallas import tpu_sc as plsc

N = 10000
E = 160000
D = 128
DA = 16
EPS = 1e-09
NORM = 1.0 / math.sqrt(20.0)

TN = 400
TE = 256
C = 128
NW = 32
NS = 16
SPLITS = [40960, 40960, 40960, 37120]
ROWS_PER_SUB = 624
ROWS_TAIL = N - NS * ROWS_PER_SUB


def _rownorm(y):
    m = jnp.mean(y, axis=1, keepdims=True)
    c = y - m
    var = jnp.sum(c * c, axis=1, keepdims=True) * (1.0 / (D - 1))
    return y / (jnp.sqrt(var) + EPS)


def _silu(z):
    return z * jax.nn.sigmoid(z)


def _stage_a_body(xn_ref, attr_ref, wbt_ref, wlt_ref, bl_ref, out_ref):
    x1 = xn_ref[...]
    x2 = attr_ref[...]
    outer = jnp.concatenate([x2[:, j:j + 1] * x1 for j in range(DA)], axis=1)
    xbi = jnp.dot(outer, wbt_ref[...], preferred_element_type=jnp.float32)
    wlt = wlt_ref[...]
    y = (jnp.dot(x1, wlt[:D], preferred_element_type=jnp.float32)
         + jnp.dot(x2, wlt[D:D + DA], preferred_element_type=jnp.float32)
         + jnp.dot(xbi, wlt[D + DA:], preferred_element_type=jnp.float32)
         + bl_ref[...])
    out_ref[...] = _rownorm(y)


def _make_gather(nch):
    per_w, rag = nch // NW, nch % NW
    npairs, odd = per_w // 2, per_w % 2

    def body(table, srci, dsti, srows, drows,
             sidx0, didx0, sbuf0, dbuf0, sidx1, didx1, sbuf1, dbuf1,
             semA, semB, semC, semD, semE, semF):
        c = lax.axis_index("c")
        s = lax.axis_index("s")
        wid = s * 2 + c

        def one(base, sidx, didx, sbuf, dbuf, semi, semg, semo):
            ci = pltpu.async_copy(srci.at[pl.ds(base, C)], sidx, semi)
            cj = pltpu.async_copy(dsti.at[pl.ds(base, C)], didx, semi)
            ci.wait()
            cj.wait()
            g1 = pltpu.async_copy(table.at[sidx], sbuf, semg)
            g2 = pltpu.async_copy(table.at[didx], dbuf, semg)
            g1.wait()
            g2.wait()
            o1 = pltpu.async_copy(sbuf, srows.at[pl.ds(base, C)], semo)
            o2 = pltpu.async_copy(dbuf, drows.at[pl.ds(base, C)], semo)
            o1.wait()
            o2.wait()

        def pair(i, carry):
            ba = (wid + NW * (2 * i)) * C
            bb = (wid + NW * (2 * i + 1)) * C
            ia1 = pltpu.async_copy(srci.at[pl.ds(ba, C)], sidx0, semA)
            ia2 = pltpu.async_copy(dsti.at[pl.ds(ba, C)], didx0, semA)
            ib1 = pltpu.async_copy(srci.at[pl.ds(bb, C)], sidx1, semB)
            ib2 = pltpu.async_copy(dsti.at[pl.ds(bb, C)], didx1, semB)
            ia1.wait()
            ia2.wait()
            ga1 = pltpu.async_copy(table.at[sidx0], sbuf0, semC)
            ga2 = pltpu.async_copy(table.at[didx0], dbuf0, semC)
            ib1.wait()
            ib2.wait()
            gb1 = pltpu.async_copy(table.at[sidx1], sbuf1, semD)
            gb2 = pltpu.async_copy(table.at[didx1], dbuf1, semD)
            ga1.wait()
            ga2.wait()
            oa1 = pltpu.async_copy(sbuf0, srows.at[pl.ds(ba, C)], semE)
            oa2 = pltpu.async_copy(dbuf0, drows.at[pl.ds(ba, C)], semE)
            gb1.wait()
            gb2.wait()
            ob1 = pltpu.async_copy(sbuf1, srows.at[pl.ds(bb, C)], semF)
            ob2 = pltpu.async_copy(dbuf1, drows.at[pl.ds(bb, C)], semF)
            oa1.wait()
            oa2.wait()
            ob1.wait()
            ob2.wait()
            return carry

        lax.fori_loop(0, npairs, pair, 0)
        if odd:
            one((wid + NW * (2 * npairs)) * C, sidx0, didx0, sbuf0, dbuf0,
                semA, semC, semE)
        if rag:
            @pl.when(wid < rag)
            def _tail():
                one((wid + NW * per_w) * C, sidx1, didx1, sbuf1, dbuf1,
                    semB, semD, semF)

    return body


def _stage_c_body(src_ref, dst_ref, a_ref, fc1_ref, bfc1_ref, fc2_ref,
                  bfc2_ref, wb2_ref, wl1_ref, bl1_ref, wbxe_ref, wl2_ref,
                  bl2_ref, out_ref):
    sT = src_ref[...].T
    dT = dst_ref[...].T
    aT = a_ref[...]
    wT = _silu(fc1_ref[...] * aT + bfc1_ref[...])
    gradT = wT * (sT - dT)
    aveT = wT * (sT + dT) * 0.5
    xbiT = jnp.zeros((D, TE), jnp.float32)
    for j0 in range(0, D, 32):
        outerT = jnp.concatenate(
            [(aveT[j:j + 1, :] * gradT).astype(jnp.bfloat16)
             for j in range(j0, j0 + 32)], axis=0)
        xbiT = xbiT + jnp.dot(wb2_ref[:, pl.ds(j0 * D, 32 * D)], outerT,
                              preferred_element_type=jnp.float32)
    wl1 = wl1_ref[...]
    gradTb = gradT.astype(jnp.bfloat16)
    aveTb = aveT.astype(jnp.bfloat16)
    xbiTb = xbiT.astype(jnp.bfloat16)
    xeT = (jnp.dot(wl1[:, :D], gradTb, preferred_element_type=jnp.float32)
           + jnp.dot(wl1[:, D:2 * D], aveTb, preferred_element_type=jnp.float32)
           + jnp.dot(wl1[:, 2 * D:], xbiTb, preferred_element_type=jnp.float32)
           + bl1_ref[...])
    xeTb = xeT.astype(jnp.bfloat16)
    xbi2T = jnp.dot(wbxe_ref[...], xeTb, preferred_element_type=jnp.float32) * aT
    wl2 = wl2_ref[...]
    xe2T = (jnp.dot(wl2[:, :D], xeTb, preferred_element_type=jnp.float32)
            + wl2[:, D:D + 1].astype(jnp.float32) * aT
            + jnp.dot(wl2[:, D + 1:], xbi2T.astype(jnp.bfloat16),
                      preferred_element_type=jnp.float32)
            + bl2_ref[...])
    m = jnp.mean(xe2T, axis=0, keepdims=True)
    cen = xe2T - m
    var = jnp.sum(cen * cen, axis=0, keepdims=True) * (1.0 / (D - 1))
    xe2T = xe2T / (jnp.sqrt(var) + EPS)
    w2T = _silu(fc2_ref[...] * aT + bfc2_ref[...])
    out_ref[...] = (w2T * xe2T * NORM).T


def _make_scatter(nch):
    per_s, rag = nch // NS, nch % NS
    npairs, odd = per_s // 2, per_s % 2

    def body(vals, dsti, srci, zrows, xn1, xn2, idxb0, vbuf0, idxb1, vbuf1,
             acc, semA, semB, semC, semD):
        c = lax.axis_index("c")
        s = lax.axis_index("s")
        pltpu.sync_copy(zrows.at[pl.ds(s * ROWS_PER_SUB, ROWS_PER_SUB)],
                        acc.at[pl.ds(s * ROWS_PER_SUB, ROWS_PER_SUB)])

        @pl.when(s == 0)
        def _ztail():
            pltpu.sync_copy(zrows.at[pl.ds(NS * ROWS_PER_SUB, ROWS_TAIL)],
                            acc.at[pl.ds(NS * ROWS_PER_SUB, ROWS_TAIL)])

        plsc.subcore_barrier()

        def run(idx_hbm):
            def one(base, idxb, vbuf, semi, semv):
                ia = pltpu.async_copy(idx_hbm.at[pl.ds(base, C)], idxb, semi)
                va = pltpu.async_copy(vals.at[pl.ds(base, C)], vbuf, semv)
                ia.wait()
                va.wait()
                pltpu.sync_copy(vbuf, acc.at[idxb], add=True)

            def pair(i, carry):
                ba = (s + NS * (2 * i)) * C
                bb = (s + NS * (2 * i + 1)) * C
                ia = pltpu.async_copy(idx_hbm.at[pl.ds(ba, C)], idxb0, semA)
                va = pltpu.async_copy(vals.at[pl.ds(ba, C)], vbuf0, semB)
                ib = pltpu.async_copy(idx_hbm.at[pl.ds(bb, C)], idxb1, semC)
                vb = pltpu.async_copy(vals.at[pl.ds(bb, C)], vbuf1, semD)
                ia.wait()
                va.wait()
                pltpu.sync_copy(vbuf0, acc.at[idxb0], add=True)
                ib.wait()
                vb.wait()
                pltpu.sync_copy(vbuf1, acc.at[idxb1], add=True)
                return carry

            lax.fori_loop(0, npairs, pair, 0)
            if odd:
                one((s + NS * (2 * npairs)) * C, idxb0, vbuf0, semA, semB)
            if rag:
                @pl.when(s < rag)
                def _tail():
                    one((s + NS * per_s) * C, idxb1, vbuf1, semC, semD)

        @pl.when(c == 0)
        def _dst():
            run(dsti)

        @pl.when(c == 1)
        def _src():
            run(srci)

        plsc.subcore_barrier()

        @pl.when(c == 0)
        def _out1():
            pltpu.sync_copy(acc.at[pl.ds(s * ROWS_PER_SUB, ROWS_PER_SUB)],
                            xn1.at[pl.ds(s * ROWS_PER_SUB, ROWS_PER_SUB)])

            @pl.when(s == 0)
            def _t1():
                pltpu.sync_copy(acc.at[pl.ds(NS * ROWS_PER_SUB, ROWS_TAIL)],
                                xn1.at[pl.ds(NS * ROWS_PER_SUB, ROWS_TAIL)])

        @pl.when(c == 1)
        def _out2():
            pltpu.sync_copy(acc.at[pl.ds(s * ROWS_PER_SUB, ROWS_PER_SUB)],
                            xn2.at[pl.ds(s * ROWS_PER_SUB, ROWS_PER_SUB)])

            @pl.when(s == 0)
            def _t2():
                pltpu.sync_copy(acc.at[pl.ds(NS * ROWS_PER_SUB, ROWS_TAIL)],
                                xn2.at[pl.ds(NS * ROWS_PER_SUB, ROWS_TAIL)])

    return body


def _stage_e_body(*refs):
    nparts = len(SPLITS)
    part_refs = refs[:2 * nparts]
    wb2_ref, wl_ref, bl_ref, out_ref = refs[2 * nparts:]
    xn1 = sum(part_refs[2 * k][...] for k in range(nparts))
    xn2 = sum(part_refs[2 * k + 1][...] for k in range(nparts))
    ddT = (xn1 - xn2).T
    smT = (xn1 + xn2).T
    xbiT = jnp.zeros((D, TN), jnp.float32)
    for j0 in range(0, D, 32):
        outerT = jnp.concatenate(
            [(smT[j:j + 1, :] * ddT).astype(jnp.bfloat16)
             for j in range(j0, j0 + 32)], axis=0)
        xbiT = xbiT + jnp.dot(wb2_ref[:, pl.ds(j0 * D, 32 * D)], outerT,
                              preferred_element_type=jnp.float32)
    wl = wl_ref[...]
    yT = (jnp.dot(wl[:, :D], ddT, preferred_element_type=jnp.float32)
          + jnp.dot(wl[:, D:2 * D], smT, preferred_element_type=jnp.float32)
          + jnp.dot(wl[:, 2 * D:], xbiT, preferred_element_type=jnp.float32)
          + bl_ref[...])
    yT = _silu(yT)
    m = jnp.mean(yT, axis=0, keepdims=True)
    cen = yT - m
    var = jnp.sum(cen * cen, axis=0, keepdims=True) * (1.0 / (D - 1))
    out_ref[...] = (yT / (jnp.sqrt(var) + EPS)).T


_GATHER_SCRATCH = [
    pltpu.VMEM((C,), jnp.int32),
    pltpu.VMEM((C,), jnp.int32),
    pltpu.VMEM((C, D), jnp.float32),
    pltpu.VMEM((C, D), jnp.float32),
    pltpu.VMEM((C,), jnp.int32),
    pltpu.VMEM((C,), jnp.int32),
    pltpu.VMEM((C, D), jnp.float32),
    pltpu.VMEM((C, D), jnp.float32),
] + [pltpu.SemaphoreType.DMA] * 6

_SCATTER_SCRATCH = [
    pltpu.VMEM((C,), jnp.int32),
    pltpu.VMEM((C, D), jnp.float32),
    pltpu.VMEM((C,), jnp.int32),
    pltpu.VMEM((C, D), jnp.float32),
    pltpu.VMEM_SHARED((N, D), jnp.float32),
] + [pltpu.SemaphoreType.DMA] * 4


def kernel(xn, xn_attr, xe_attr, xe_src, xe_dst, Wb_xn, Wl_xn, bl_xn,
           W_fc1, b_fc1, Wb_n2e, Wl_n2e, bl_n2e, Wb_xe, Wl_xe, bl_xe,
           W_fc2, b_fc2, Wb_e2n, Wl_e2n, bl_e2n):
    f32 = jnp.float32
    xe_src = xe_src.astype(jnp.int32)
    xe_dst = xe_dst.astype(jnp.int32)

    wbt_xn = jnp.transpose(Wb_xn, (2, 1, 0)).reshape(DA * D, D)
    wb2_n2e = jnp.transpose(Wb_n2e, (0, 2, 1)).reshape(D, D * D).astype(jnp.bfloat16)
    wb2_e2n = jnp.transpose(Wb_e2n, (0, 2, 1)).reshape(D, D * D).astype(jnp.bfloat16)
    wlt_xn = Wl_xn.T
    wl_n2e_b = Wl_n2e.astype(jnp.bfloat16)
    wl_xe_b = Wl_xe.astype(jnp.bfloat16)
    wbxe0 = Wb_xe[:, :, 0].astype(jnp.bfloat16)
    aT_edge = xe_attr.T
    fc1c = W_fc1
    fc2c = W_fc2
    bfc1c = b_fc1.reshape(D, 1)
    bfc2c = b_fc2.reshape(D, 1)
    bl_xn2 = bl_xn.reshape(1, D)
    bl_n2ec = bl_n2e.reshape(D, 1)
    bl_xec = bl_xe.reshape(D, 1)

    xnm = pl.pallas_call(
        _stage_a_body,
        grid=(N // TN,),
        in_specs=[
            pl.BlockSpec((TN, D), lambda i: (i, 0)),
            pl.BlockSpec((TN, DA), lambda i: (i, 0)),
            pl.BlockSpec((DA * D, D), lambda i: (0, 0)),
            pl.BlockSpec((2 * D + DA, D), lambda i: (0, 0)),
            pl.BlockSpec((1, D), lambda i: (0, 0)),
        ],
        out_specs=pl.BlockSpec((TN, D), lambda i: (i, 0)),
        out_shape=jax.ShapeDtypeStruct((N, D), f32),
    )(xn, xn_attr, wbt_xn, wlt_xn, bl_xn2)

    mesh = plsc.VectorSubcoreMesh(core_axis_name="c", subcore_axis_name="s")

    def gather(nch, srci, dsti):
        return pl.kernel(
            _make_gather(nch),
            out_type=[jax.ShapeDtypeStruct((nch * C, D), f32),
                      jax.ShapeDtypeStruct((nch * C, D), f32)],
            mesh=mesh,
            scratch_types=_GATHER_SCRATCH,
        )(xnm, srci, dsti)

    def edge_compute(srows, drows, aT, ne):
        return pl.pallas_call(
            _stage_c_body,
            grid=(ne // TE,),
            in_specs=[
                pl.BlockSpec((TE, D), lambda i: (i, 0)),
                pl.BlockSpec((TE, D), lambda i: (i, 0)),
                pl.BlockSpec((1, TE), lambda i: (0, i)),
                pl.BlockSpec((D, 1), lambda i: (0, 0)),
                pl.BlockSpec((D, 1), lambda i: (0, 0)),
                pl.BlockSpec((D, 1), lambda i: (0, 0)),
                pl.BlockSpec((D, 1), lambda i: (0, 0)),
                pl.BlockSpec((D, D * D), lambda i: (0, 0)),
                pl.BlockSpec((D, 3 * D), lambda i: (0, 0)),
                pl.BlockSpec((D, 1), lambda i: (0, 0)),
                pl.BlockSpec((D, D), lambda i: (0, 0)),
                pl.BlockSpec((D, 2 * D + 1), lambda i: (0, 0)),
                pl.BlockSpec((D, 1), lambda i: (0, 0)),
            ],
            out_specs=pl.BlockSpec((TE, D), lambda i: (i, 0)),
            out_shape=jax.ShapeDtypeStruct((ne, D), f32),
        )(srows, drows, aT, fc1c, bfc1c, fc2c, bfc2c, wb2_n2e, wl_n2e_b,
          bl_n2ec, wbxe0, wl_xe_b, bl_xec)

    zrows = jnp.zeros((N, D), f32)

    def scatter(nch, vals, dsti, srci):
        return pl.kernel(
            _make_scatter(nch),
            out_type=[jax.ShapeDtypeStruct((N, D), f32),
                      jax.ShapeDtypeStruct((N, D), f32)],
            mesh=mesh,
            scratch_types=_SCATTER_SCRATCH,
        )(vals, dsti, srci, zrows)

    bounds = [0]
    for ne in SPLITS:
        bounds.append(bounds[-1] + ne)
    parts = []
    for k in range(len(SPLITS)):
        lo, ne = bounds[k], SPLITS[k]
        parts.append((xe_src[lo:lo + ne], xe_dst[lo:lo + ne],
                      aT_edge[:, lo:lo + ne], ne))

    rows = [gather(ne // C, s, d) for (s, d, _, ne) in parts]
    vals = [edge_compute(r[0], r[1], a, ne)
            for r, (_, _, a, ne) in zip(rows, parts)]
    sums = [scatter(ne // C, v, d, s)
            for v, (s, d, _, ne) in zip(vals, parts)]

    out = pl.pallas_call(
        _stage_e_body,
        grid=(N // TN,),
        in_specs=(
            [pl.BlockSpec((TN, D), lambda i: (i, 0))] * (2 * len(SPLITS)) + [
                pl.BlockSpec((D, D * D), lambda i: (0, 0)),
                pl.BlockSpec((D, 3 * D), lambda i: (0, 0)),
                pl.BlockSpec((D, 1), lambda i: (0, 0)),
            ]),
        out_specs=pl.BlockSpec((TN, D), lambda i: (i, 0)),
        out_shape=jax.ShapeDtypeStruct((N, D), f32),
    )(*[x for pair in sums for x in pair],
      wb2_e2n, Wl_e2n, bl_e2n.reshape(D, 1))
    return out
